# Initial kernel scaffold; baseline (speedup 1.0000x reference)
import jax
import jax.numpy as jnp
from jax import lax
from jax.experimental import pallas as pl
from jax.experimental.pallas import tpu as pltpu

N, E, B = 10000, 160000, 16
DV, DE, DG = 256, 16, 64
HE, OE = 512, 256
HV, OV = 512, 256
HG, OG = 512, 128

VBLK = 400
EBLK = 640

_f32 = jnp.float32


def _full(shape):
    return pl.BlockSpec(shape, lambda i: tuple(0 for _ in shape))


def _k2_body(vb, bb, g, A, Bm, Av, Cg, Cv, be1, bv1, P_o, Q_o, V1_o):
    G1 = jnp.dot(g[:, :], Cg[:, :], preferred_element_type=_f32) + be1[:, :]
    G2 = jnp.dot(g[:, :], Cv[:, :], preferred_element_type=_f32) + bv1[:, :]
    b = bb[0, 0, :]
    oh = (b[:, None] == lax.broadcasted_iota(jnp.int32, (VBLK, B), 1)).astype(_f32)
    P_o[:, :] = (jnp.dot(vb[:, :], A[:, :], preferred_element_type=_f32)
                 + jnp.dot(oh, G1, preferred_element_type=_f32))
    Q_o[:, :] = jnp.dot(vb[:, :], Bm[:, :], preferred_element_type=_f32)
    V1_o[:, :] = (jnp.dot(vb[:, :], Av[:, :], preferred_element_type=_f32)
                  + jnp.dot(oh, G2, preferred_element_type=_f32))


def _k2(v_attr, batch3, g, A, Bm, Av, Cg, Cv, be1, bv1):
    nb = N // VBLK
    return pl.pallas_call(
        _k2_body,
        grid=(nb,),
        in_specs=[
            pl.BlockSpec((VBLK, DV), lambda i: (i, 0)),
            pl.BlockSpec((1, 1, VBLK), lambda i: (i, 0, 0)),
            _full((B, DG)), _full((DV, HE)), _full((DV, HE)), _full((DV, HV)),
            _full((DG, HE)), _full((DG, HV)), _full((1, HE)), _full((1, HV)),
        ],
        out_specs=[
            pl.BlockSpec((VBLK, HE), lambda i: (i, 0)),
            pl.BlockSpec((VBLK, HE), lambda i: (i, 0)),
            pl.BlockSpec((VBLK, HV), lambda i: (i, 0)),
        ],
        out_shape=[
            jax.ShapeDtypeStruct((N, HE), _f32),
            jax.ShapeDtypeStruct((N, HE), _f32),
            jax.ShapeDtypeStruct((N, HV), _f32),
        ],
    )(v_attr, batch3, g, A, Bm, Av, Cg, Cv, be1, bv1)


def _k3_body(zb, eb, Ce, We2, be2, out_o):
    h = jnp.maximum(zb[:, :] + jnp.dot(eb[:, :], Ce[:, :], preferred_element_type=_f32), 0.0)
    out_o[:, :] = jnp.dot(h, We2[:, :], preferred_element_type=_f32) + be2[:, :]


def _k3(Zg, e_attr, Ce, We2, be2):
    nb = E // EBLK
    return pl.pallas_call(
        _k3_body,
        grid=(nb,),
        in_specs=[
            pl.BlockSpec((EBLK, HE), lambda i: (i, 0)),
            pl.BlockSpec((EBLK, DE), lambda i: (i, 0)),
            _full((DE, HE)), _full((HE, OE)), _full((1, OE)),
        ],
        out_specs=pl.BlockSpec((EBLK, OE), lambda i: (i, 0)),
        out_shape=jax.ShapeDtypeStruct((E, OE), _f32),
    )(Zg, e_attr, Ce, We2, be2)


def _k4_body(v1b, mnb, sb, mxb, cb, Wmn, Wme, Wsm, Wmx, Wv2, bv2, out_o):
    c = cb[:, 0:1]
    has = c > 0.0
    mn = jnp.where(has, mnb[:, :], 0.0)
    mx = jnp.where(has, mxb[:, :], 0.0)
    mean = sb[:, :] * (1.0 / jnp.maximum(c, 1.0))
    z = (v1b[:, :]
         + jnp.dot(mn, Wmn[:, :], preferred_element_type=_f32)
         + jnp.dot(mean, Wme[:, :], preferred_element_type=_f32)
         + jnp.dot(sb[:, :], Wsm[:, :], preferred_element_type=_f32)
         + jnp.dot(mx, Wmx[:, :], preferred_element_type=_f32))
    out_o[:, :] = jnp.dot(jnp.maximum(z, 0.0), Wv2[:, :], preferred_element_type=_f32) + bv2[:, :]


def _k4(V1, MN, S, MX, CNT, Wmn, Wme, Wsm, Wmx, Wv2, bv2):
    nb = N // VBLK
    return pl.pallas_call(
        _k4_body,
        grid=(nb,),
        in_specs=[
            pl.BlockSpec((VBLK, HV), lambda i: (i, 0)),
            pl.BlockSpec((VBLK, OE), lambda i: (i, 0)),
            pl.BlockSpec((VBLK, OE), lambda i: (i, 0)),
            pl.BlockSpec((VBLK, OE), lambda i: (i, 0)),
            pl.BlockSpec((VBLK, 16), lambda i: (i, 0)),
            _full((OE, HV)), _full((OE, HV)), _full((OE, HV)), _full((OE, HV)),
            _full((HV, OV)), _full((1, OV)),
        ],
        out_specs=pl.BlockSpec((VBLK, OV), lambda i: (i, 0)),
        out_shape=jax.ShapeDtypeStruct((N, OV), _f32),
    )(V1, MN, S, MX, CNT, Wmn, Wme, Wsm, Wmx, Wv2, bv2)


def _k5_body(mnb, sb, mxb, cb, vob, bb, gg, wgg, wem, wee, wes, wex, wvm, wve,
             wvs, wvx, wg2, bg1r, bg2r, out_o,
             emin_s, emax_s, esum_s, ecnt_s, vmin_s, vmax_s, vsum_s, vcnt_s):
    pid = pl.program_id(0)
    nb = pl.num_programs(0)

    @pl.when(pid == 0)
    def _init():
        emin_s[:, :] = jnp.full((B, OE), jnp.inf, _f32)
        emax_s[:, :] = jnp.full((B, OE), -jnp.inf, _f32)
        esum_s[:, :] = jnp.zeros((B, OE), _f32)
        ecnt_s[:, :] = jnp.zeros((B, 128), _f32)
        vmin_s[:, :] = jnp.full((B, OV), jnp.inf, _f32)
        vmax_s[:, :] = jnp.full((B, OV), -jnp.inf, _f32)
        vsum_s[:, :] = jnp.zeros((B, OV), _f32)
        vcnt_s[:, :] = jnp.zeros((B, 128), _f32)

    b = bb[0, 0, :]
    c1 = cb[:, 0]
    mn = mnb[:, :]
    mx = mxb[:, :]
    s = sb[:, :]
    vo = vob[:, :]
    for j in range(B):
        mv = b == j
        me = (mv & (c1 > 0.0))[:, None]
        mvn = mv[:, None]
        emin_s[j, :] = jnp.minimum(emin_s[j, :], jnp.min(jnp.where(me, mn, jnp.inf), axis=0))
        emax_s[j, :] = jnp.maximum(emax_s[j, :], jnp.max(jnp.where(me, mx, -jnp.inf), axis=0))
        esum_s[j, :] = esum_s[j, :] + jnp.sum(jnp.where(mvn, s, 0.0), axis=0)
        ecnt_s[j, :] = ecnt_s[j, :] + jnp.sum(jnp.where(mv, c1, 0.0))
        vmin_s[j, :] = jnp.minimum(vmin_s[j, :], jnp.min(jnp.where(mvn, vo, jnp.inf), axis=0))
        vmax_s[j, :] = jnp.maximum(vmax_s[j, :], jnp.max(jnp.where(mvn, vo, -jnp.inf), axis=0))
        vsum_s[j, :] = vsum_s[j, :] + jnp.sum(jnp.where(mvn, vo, 0.0), axis=0)
        vcnt_s[j, :] = vcnt_s[j, :] + jnp.sum(mv.astype(_f32))

    @pl.when(pid == nb - 1)
    def _final():
        ec = ecnt_s[:, 0:1]
        vc = vcnt_s[:, 0:1]
        emn = jnp.where(ec > 0.0, emin_s[:, :], 0.0)
        emx = jnp.where(ec > 0.0, emax_s[:, :], 0.0)
        eme = esum_s[:, :] * (1.0 / jnp.maximum(ec, 1.0))
        vmn = jnp.where(vc > 0.0, vmin_s[:, :], 0.0)
        vmx = jnp.where(vc > 0.0, vmax_s[:, :], 0.0)
        vme = vsum_s[:, :] * (1.0 / jnp.maximum(vc, 1.0))
        dot = lambda a, w: jnp.dot(a, w[:, :], preferred_element_type=_f32)
        z = (dot(gg[:, :], wgg) + dot(emn, wem) + dot(eme, wee)
             + dot(esum_s[:, :], wes) + dot(emx, wex) + dot(vmn, wvm)
             + dot(vme, wve) + dot(vsum_s[:, :], wvs) + dot(vmx, wvx)
             + bg1r[:, :])
        out_o[:, :] = dot(jnp.maximum(z, 0.0), wg2) + bg2r[:, :]


def _k5(MN, S, MX, CNT, v_out, batch3, g, Wg_slices, bg1, Wg2, bg2):
    nb = N // VBLK
    return pl.pallas_call(
        _k5_body,
        grid=(nb,),
        in_specs=[
            pl.BlockSpec((VBLK, OE), lambda i: (i, 0)),
            pl.BlockSpec((VBLK, OE), lambda i: (i, 0)),
            pl.BlockSpec((VBLK, OE), lambda i: (i, 0)),
            pl.BlockSpec((VBLK, 16), lambda i: (i, 0)),
            pl.BlockSpec((VBLK, OV), lambda i: (i, 0)),
            pl.BlockSpec((1, 1, VBLK), lambda i: (i, 0, 0)),
            _full((B, DG)),
            _full((DG, HG)), _full((OE, HG)), _full((OE, HG)), _full((OE, HG)),
            _full((OE, HG)), _full((OV, HG)), _full((OV, HG)), _full((OV, HG)),
            _full((OV, HG)), _full((HG, OG)), _full((1, HG)), _full((1, OG)),
        ],
        out_specs=pl.BlockSpec((B, OG), lambda i: (0, 0)),
        out_shape=jax.ShapeDtypeStruct((B, OG), _f32),
        scratch_shapes=[
            pltpu.VMEM((B, OE), _f32), pltpu.VMEM((B, OE), _f32),
            pltpu.VMEM((B, OE), _f32), pltpu.VMEM((B, 128), _f32),
            pltpu.VMEM((B, OV), _f32), pltpu.VMEM((B, OV), _f32),
            pltpu.VMEM((B, OV), _f32), pltpu.VMEM((B, 128), _f32),
        ],
    )(MN, S, MX, CNT, v_out, batch3, g, *Wg_slices, Wg2, bg1, bg2)


def kernel(v_attr, e_attr, g, We1, be1, We2, be2, Wv1, bv1, Wv2, bv2,
           Wg1, bg1, Wg2, bg2, edgeij_pair, batch):
    src = edgeij_pair[0].astype(jnp.int32)
    dst = edgeij_pair[1].astype(jnp.int32)
    batch = batch.astype(jnp.int32)
    batch3 = batch.reshape(N // VBLK, 1, VBLK)

    A, Bm, Ce, Cg = We1[0:256], We1[256:512], We1[512:528], We1[528:592]
    Av, Wmn, Wme, Wsm, Wmx, Cv = (Wv1[0:256], Wv1[256:512], Wv1[512:768],
                                  Wv1[768:1024], Wv1[1024:1280], Wv1[1280:1344])
    Wg_slices = (Wg1[0:64], Wg1[64:320], Wg1[320:576], Wg1[576:832],
                 Wg1[832:1088], Wg1[1088:1344], Wg1[1344:1600],
                 Wg1[1600:1856], Wg1[1856:2112])
    be1r, bv1r, be2r, bv2r = (be1.reshape(1, HE), bv1.reshape(1, HV),
                              be2.reshape(1, OE), bv2.reshape(1, OV))
    bg1r, bg2r = bg1.reshape(1, HG), bg2.reshape(1, OG)

    P2, Q, V1 = _k2(v_attr, batch3, g, A, Bm, Av, Cg, Cv, be1r, bv1r)

    Zg = jnp.take(P2, src, axis=0) + jnp.take(Q, dst, axis=0)

    e_out = _k3(Zg, e_attr, Ce, We2, be2r)

    S = jax.ops.segment_sum(e_out, src, num_segments=N)
    CNTv = jax.ops.segment_sum(jnp.ones((E,), _f32), src, num_segments=N)
    MN = jax.ops.segment_min(e_out, src, num_segments=N)
    MX = jax.ops.segment_max(e_out, src, num_segments=N)
    CNT = jnp.broadcast_to(CNTv[:, None], (N, 16))

    v_out = _k4(V1, MN, S, MX, CNT, Wmn, Wme, Wsm, Wmx, Wv2, bv2r)

    g_out = _k5(MN, S, MX, CNT, v_out, batch3, g, Wg_slices, bg1r, Wg2, bg2r)

    return (e_out, v_out, g_out)


# speedup vs baseline: 2.6490x; 2.6490x over previous
import jax
import jax.numpy as jnp
from jax import lax
from jax.experimental import pallas as pl
from jax.experimental.pallas import tpu as pltpu
from jax.experimental.pallas import tpu_sc as plsc

N, E, B = 10000, 160000, 16
DV, DE, DG = 256, 16, 64
HE, OE = 512, 256
HV, OV = 512, 256
HG, OG = 512, 128

VBLK = 400
EBLK = 640

_f32 = jnp.float32
_i32 = jnp.int32

_NC, _NS, _LL = 2, 16, 16
_NW = _NC * _NS


def _full(shape):
    return pl.BlockSpec(shape, lambda i: tuple(0 for _ in shape))


def _k2_body(vb, bb, g, A, Bm, Av, Cg, Cv, be1, bv1, P_o, Q_o, V1_o):
    G1 = jnp.dot(g[:, :], Cg[:, :], preferred_element_type=_f32) + be1[:, :]
    G2 = jnp.dot(g[:, :], Cv[:, :], preferred_element_type=_f32) + bv1[:, :]
    bcol = bb[:, :]
    oh = (bcol == lax.broadcasted_iota(_i32, (VBLK, B), 1).astype(_f32)).astype(_f32)
    P_o[:, :] = (jnp.dot(vb[:, :], A[:, :], preferred_element_type=_f32)
                 + jnp.dot(oh, G1, preferred_element_type=_f32))
    Q_o[:, :] = jnp.dot(vb[:, :], Bm[:, :], preferred_element_type=_f32)
    V1_o[:, :] = (jnp.dot(vb[:, :], Av[:, :], preferred_element_type=_f32)
                  + jnp.dot(oh, G2, preferred_element_type=_f32))


def _k2(v_attr, batchcol, g, A, Bm, Av, Cg, Cv, be1, bv1):
    nb = N // VBLK
    return pl.pallas_call(
        _k2_body,
        grid=(nb,),
        in_specs=[
            pl.BlockSpec((VBLK, DV), lambda i: (i, 0)),
            pl.BlockSpec((VBLK, 1), lambda i: (i, 0)),
            _full((B, DG)), _full((DV, HE)), _full((DV, HE)), _full((DV, HV)),
            _full((DG, HE)), _full((DG, HV)), _full((1, HE)), _full((1, HV)),
        ],
        out_specs=[
            pl.BlockSpec((VBLK, HE), lambda i: (i, 0)),
            pl.BlockSpec((VBLK, HE), lambda i: (i, 0)),
            pl.BlockSpec((VBLK, HV), lambda i: (i, 0)),
        ],
        out_shape=[
            jax.ShapeDtypeStruct((N, HE), _f32),
            jax.ShapeDtypeStruct((N, HE), _f32),
            jax.ShapeDtypeStruct((N, HV), _f32),
        ],
    )(v_attr, batchcol, g, A, Bm, Av, Cg, Cv, be1, bv1)


_EW = E // _NW
_ACH = 40
_ANCH = _EW // _ACH


def _sc_gather_add(P2, Q, src, dst):
    def body(p_hbm, q_hbm, src_hbm, dst_hbm, zg_hbm,
             sidx, didx, bufp, bufq, sem1, sem2):
        wid = lax.axis_index("s") * _NC + lax.axis_index("c")
        base = wid * _EW

        def chunk(ci, carry):
            off = base + ci * _ACH
            pltpu.sync_copy(src_hbm.at[pl.ds(off, _ACH)], sidx)
            pltpu.sync_copy(dst_hbm.at[pl.ds(off, _ACH)], didx)
            cp = pltpu.async_copy(p_hbm.at[sidx], bufp, sem1)
            cq = pltpu.async_copy(q_hbm.at[didx], bufq, sem2)
            cp.wait()
            cq.wait()

            def row(r, c2):
                for k in range(HE // _LL):
                    sl = pl.ds(k * _LL, _LL)
                    bufp[r, sl] = bufp[r, sl] + bufq[r, sl]
                return c2
            lax.fori_loop(0, _ACH, row, 0)
            pltpu.sync_copy(bufp, zg_hbm.at[pl.ds(off, _ACH)])
            return carry
        lax.fori_loop(0, _ANCH, chunk, 0)

    kern = pl.kernel(
        body,
        out_type=jax.ShapeDtypeStruct((E, HE), _f32),
        mesh=plsc.VectorSubcoreMesh(core_axis_name="c", subcore_axis_name="s"),
        scratch_types=[
            pltpu.VMEM((_ACH,), _i32),
            pltpu.VMEM((_ACH,), _i32),
            pltpu.VMEM((_ACH, HE), _f32),
            pltpu.VMEM((_ACH, HE), _f32),
            pltpu.SemaphoreType.DMA,
            pltpu.SemaphoreType.DMA,
        ],
    )
    return kern(P2, Q, src, dst)


def _k3_body(zb, eb, Ce, We2, be2, out_o):
    h = jnp.maximum(zb[:, :] + jnp.dot(eb[:, :], Ce[:, :], preferred_element_type=_f32), 0.0)
    out_o[:, :] = jnp.dot(h, We2[:, :], preferred_element_type=_f32) + be2[:, :]


def _k3(Zg, e_attr, Ce, We2, be2):
    nb = E // EBLK
    return pl.pallas_call(
        _k3_body,
        grid=(nb,),
        in_specs=[
            pl.BlockSpec((EBLK, HE), lambda i: (i, 0)),
            pl.BlockSpec((EBLK, DE), lambda i: (i, 0)),
            _full((DE, HE)), _full((HE, OE)), _full((1, OE)),
        ],
        out_specs=pl.BlockSpec((EBLK, OE), lambda i: (i, 0)),
        out_shape=jax.ShapeDtypeStruct((E, OE), _f32),
    )(Zg, e_attr, Ce, We2, be2)


EBLK6 = 2000


def _k6_body(srcb, eb, s_o, mn_o, mx_o, c_o):
    pid = pl.program_id(0)

    @pl.when(pid == 0)
    def _init():
        s_o[:, :] = jnp.zeros((N, OE), _f32)
        mn_o[:, :] = jnp.full((N, OE), jnp.inf, _f32)
        mx_o[:, :] = jnp.full((N, OE), -jnp.inf, _f32)
        c_o[:, :] = jnp.zeros((N, 16), _f32)

    def edge(i, carry):
        v = srcb[0, 0, i]
        row = eb[pl.ds(i, 1), :]
        s_o[pl.ds(v, 1), :] += row
        mn_o[pl.ds(v, 1), :] = jnp.minimum(mn_o[pl.ds(v, 1), :], row)
        mx_o[pl.ds(v, 1), :] = jnp.maximum(mx_o[pl.ds(v, 1), :], row)
        c_o[pl.ds(v, 1), :] += 1.0
        return carry
    lax.fori_loop(0, EBLK6, edge, 0)


def _k6(e_out, src2):
    nb = E // EBLK6
    return pl.pallas_call(
        _k6_body,
        grid=(nb,),
        in_specs=[
            pl.BlockSpec((1, 1, EBLK6), lambda i: (i, 0, 0), memory_space=pltpu.SMEM),
            pl.BlockSpec((EBLK6, OE), lambda i: (i, 0)),
        ],
        out_specs=[
            pl.BlockSpec((N, OE), lambda i: (0, 0)),
            pl.BlockSpec((N, OE), lambda i: (0, 0)),
            pl.BlockSpec((N, OE), lambda i: (0, 0)),
            pl.BlockSpec((N, 16), lambda i: (0, 0)),
        ],
        out_shape=[
            jax.ShapeDtypeStruct((N, OE), _f32),
            jax.ShapeDtypeStruct((N, OE), _f32),
            jax.ShapeDtypeStruct((N, OE), _f32),
            jax.ShapeDtypeStruct((N, 16), _f32),
        ],
    )(src2, e_out)


def _k4_body(v1b, mnb, sb, mxb, cb, Wmn, Wme, Wsm, Wmx, Wv2, bv2, out_o):
    c = cb[:, 0:1]
    has = c > 0.0
    mn = jnp.where(has, mnb[:, :], 0.0)
    mx = jnp.where(has, mxb[:, :], 0.0)
    mean = sb[:, :] * (1.0 / jnp.maximum(c, 1.0))
    z = (v1b[:, :]
         + jnp.dot(mn, Wmn[:, :], preferred_element_type=_f32)
         + jnp.dot(mean, Wme[:, :], preferred_element_type=_f32)
         + jnp.dot(sb[:, :], Wsm[:, :], preferred_element_type=_f32)
         + jnp.dot(mx, Wmx[:, :], preferred_element_type=_f32))
    out_o[:, :] = jnp.dot(jnp.maximum(z, 0.0), Wv2[:, :], preferred_element_type=_f32) + bv2[:, :]


def _k4(V1, MN, S, MX, CNT, Wmn, Wme, Wsm, Wmx, Wv2, bv2):
    nb = N // VBLK
    return pl.pallas_call(
        _k4_body,
        grid=(nb,),
        in_specs=[
            pl.BlockSpec((VBLK, HV), lambda i: (i, 0)),
            pl.BlockSpec((VBLK, OE), lambda i: (i, 0)),
            pl.BlockSpec((VBLK, OE), lambda i: (i, 0)),
            pl.BlockSpec((VBLK, OE), lambda i: (i, 0)),
            pl.BlockSpec((VBLK, 16), lambda i: (i, 0)),
            _full((OE, HV)), _full((OE, HV)), _full((OE, HV)), _full((OE, HV)),
            _full((HV, OV)), _full((1, OV)),
        ],
        out_specs=pl.BlockSpec((VBLK, OV), lambda i: (i, 0)),
        out_shape=jax.ShapeDtypeStruct((N, OV), _f32),
    )(V1, MN, S, MX, CNT, Wmn, Wme, Wsm, Wmx, Wv2, bv2)


def _k5_body(mnb, sb, mxb, cb, vob, bcolb, b3b, gg, wgg, wem, wee, wes, wex,
             wvm, wve, wvs, wvx, wg2, bg1r, bg2r, out_o,
             emin_s, emax_s, esum_s, ecnt_s, vmin_s, vmax_s, vsum_s, vcnt_s):
    pid = pl.program_id(0)
    nb = pl.num_programs(0)

    @pl.when(pid == 0)
    def _init():
        emin_s[:, :] = jnp.full((B, OE), jnp.inf, _f32)
        emax_s[:, :] = jnp.full((B, OE), -jnp.inf, _f32)
        esum_s[:, :] = jnp.zeros((B, OE), _f32)
        ecnt_s[:, :] = jnp.zeros((B, 16), _f32)
        vmin_s[:, :] = jnp.full((B, OV), jnp.inf, _f32)
        vmax_s[:, :] = jnp.full((B, OV), -jnp.inf, _f32)
        vsum_s[:, :] = jnp.zeros((B, OV), _f32)
        vcnt_s[:, :] = jnp.zeros((B, 16), _f32)

    bcol = bcolb[:, :]
    brow = b3b[0, :, :]
    ohT = (brow == lax.broadcasted_iota(_i32, (B, VBLK), 0)).astype(_f32)
    mn = mnb[:, :]
    mx = mxb[:, :]
    s = sb[:, :]
    vo = vob[:, :]
    ones = jnp.full((VBLK, 16), 1.0, _f32)

    esum_s[:, :] = esum_s[:, :] + jnp.dot(ohT, s, preferred_element_type=_f32)
    ecnt_s[:, :] = ecnt_s[:, :] + jnp.dot(ohT, cb[:, :], preferred_element_type=_f32)
    vsum_s[:, :] = vsum_s[:, :] + jnp.dot(ohT, vo, preferred_element_type=_f32)
    vcnt_s[:, :] = vcnt_s[:, :] + jnp.dot(ohT, ones, preferred_element_type=_f32)

    for j in range(B):
        mv = bcol == float(j)
        emin_s[j:j + 1, :] = jnp.minimum(
            emin_s[j:j + 1, :], jnp.min(jnp.where(mv, mn, jnp.inf), axis=0, keepdims=True))
        emax_s[j:j + 1, :] = jnp.maximum(
            emax_s[j:j + 1, :], jnp.max(jnp.where(mv, mx, -jnp.inf), axis=0, keepdims=True))
        vmin_s[j:j + 1, :] = jnp.minimum(
            vmin_s[j:j + 1, :], jnp.min(jnp.where(mv, vo, jnp.inf), axis=0, keepdims=True))
        vmax_s[j:j + 1, :] = jnp.maximum(
            vmax_s[j:j + 1, :], jnp.max(jnp.where(mv, vo, -jnp.inf), axis=0, keepdims=True))

    @pl.when(pid == nb - 1)
    def _final():
        ec = ecnt_s[:, 0:1]
        vc = vcnt_s[:, 0:1]
        emn = jnp.where(ec > 0.0, emin_s[:, :], 0.0)
        emx = jnp.where(ec > 0.0, emax_s[:, :], 0.0)
        eme = esum_s[:, :] * (1.0 / jnp.maximum(ec, 1.0))
        vmn = jnp.where(vc > 0.0, vmin_s[:, :], 0.0)
        vmx = jnp.where(vc > 0.0, vmax_s[:, :], 0.0)
        vme = vsum_s[:, :] * (1.0 / jnp.maximum(vc, 1.0))
        dot = lambda a, w: jnp.dot(a, w[:, :], preferred_element_type=_f32)
        z = (dot(gg[:, :], wgg) + dot(emn, wem) + dot(eme, wee)
             + dot(esum_s[:, :], wes) + dot(emx, wex) + dot(vmn, wvm)
             + dot(vme, wve) + dot(vsum_s[:, :], wvs) + dot(vmx, wvx)
             + bg1r[:, :])
        out_o[:, :] = dot(jnp.maximum(z, 0.0), wg2) + bg2r[:, :]


def _k5(MN, S, MX, CNT, v_out, batchcol, batch3, g, Wg_slices, bg1, Wg2, bg2):
    nb = N // VBLK
    return pl.pallas_call(
        _k5_body,
        grid=(nb,),
        in_specs=[
            pl.BlockSpec((VBLK, OE), lambda i: (i, 0)),
            pl.BlockSpec((VBLK, OE), lambda i: (i, 0)),
            pl.BlockSpec((VBLK, OE), lambda i: (i, 0)),
            pl.BlockSpec((VBLK, 16), lambda i: (i, 0)),
            pl.BlockSpec((VBLK, OV), lambda i: (i, 0)),
            pl.BlockSpec((VBLK, 1), lambda i: (i, 0)),
            pl.BlockSpec((1, 1, VBLK), lambda i: (i, 0, 0)),
            _full((B, DG)),
            _full((DG, HG)), _full((OE, HG)), _full((OE, HG)), _full((OE, HG)),
            _full((OE, HG)), _full((OV, HG)), _full((OV, HG)), _full((OV, HG)),
            _full((OV, HG)), _full((HG, OG)), _full((1, HG)), _full((1, OG)),
        ],
        out_specs=pl.BlockSpec((B, OG), lambda i: (0, 0)),
        out_shape=jax.ShapeDtypeStruct((B, OG), _f32),
        scratch_shapes=[
            pltpu.VMEM((B, OE), _f32), pltpu.VMEM((B, OE), _f32),
            pltpu.VMEM((B, OE), _f32), pltpu.VMEM((B, 16), _f32),
            pltpu.VMEM((B, OV), _f32), pltpu.VMEM((B, OV), _f32),
            pltpu.VMEM((B, OV), _f32), pltpu.VMEM((B, 16), _f32),
        ],
    )(MN, S, MX, CNT, v_out, batchcol, batch3, g, *Wg_slices, Wg2, bg1, bg2)


def kernel(v_attr, e_attr, g, We1, be1, We2, be2, Wv1, bv1, Wv2, bv2,
           Wg1, bg1, Wg2, bg2, edgeij_pair, batch):
    src = edgeij_pair[0].astype(_i32)
    dst = edgeij_pair[1].astype(_i32)
    batch = batch.astype(_i32)
    batchcol = batch.astype(_f32).reshape(N, 1)
    batch3 = batch.reshape(N // VBLK, 1, VBLK)

    A, Bm, Ce, Cg = We1[0:256], We1[256:512], We1[512:528], We1[528:592]
    Av, Wmn, Wme, Wsm, Wmx, Cv = (Wv1[0:256], Wv1[256:512], Wv1[512:768],
                                  Wv1[768:1024], Wv1[1024:1280], Wv1[1280:1344])
    Wg_slices = (Wg1[0:64], Wg1[64:320], Wg1[320:576], Wg1[576:832],
                 Wg1[832:1088], Wg1[1088:1344], Wg1[1344:1600],
                 Wg1[1600:1856], Wg1[1856:2112])
    be1r, bv1r, be2r, bv2r = (be1.reshape(1, HE), bv1.reshape(1, HV),
                              be2.reshape(1, OE), bv2.reshape(1, OV))
    bg1r, bg2r = bg1.reshape(1, HG), bg2.reshape(1, OG)

    P2, Q, V1 = _k2(v_attr, batchcol, g, A, Bm, Av, Cg, Cv, be1r, bv1r)

    Zg = _sc_gather_add(P2, Q, src, dst)

    e_out = _k3(Zg, e_attr, Ce, We2, be2r)

    S, MN, MX, CNT = _k6(e_out, src.reshape(E // EBLK6, 1, EBLK6))

    v_out = _k4(V1, MN, S, MX, CNT, Wmn, Wme, Wsm, Wmx, Wv2, bv2r)

    g_out = _k5(MN, S, MX, CNT, v_out, batchcol, batch3, g, Wg_slices,
                bg1r, Wg2, bg2r)

    return (e_out, v_out, g_out)
